# Initial kernel scaffold; baseline (speedup 1.0000x reference)
import jax
import jax.numpy as jnp
from jax import lax
from jax.experimental import pallas as pl
from jax.experimental.pallas import tpu as pltpu
from jax.experimental.pallas import tpu_sc as plsc

jax.config.update("jax_enable_x64", True)

_NUM_ENTRIES = 65536
_NUM_HINTS = 4096
_MAX_SUBSET = 256
_ROW = 16
_NC = 2
_NS = 16
_NW = _NC * _NS
_HPT = _NUM_HINTS // _NW
_CHUNK = 128
_NCHUNK = _MAX_SUBSET // _CHUNK


def _sc_body(table, idx, msk, out, idx_v, msk_v, rows0_v, rows1_v, out_v,
             row0_v, sem):
    wid = lax.axis_index("s") * _NC + lax.axis_index("c")
    base = wid * _HPT
    pltpu.sync_copy(idx.at[pl.ds(base, _HPT)], idx_v)
    pltpu.sync_copy(msk.at[pl.ds(base, _HPT)], msk_v)
    pltpu.sync_copy(table.at[0], row0_v)
    row0 = row0_v[...]

    def hint(h, carry):
        cnt = jnp.zeros((16,), jnp.int32)
        for c in range(_NCHUNK):
            for o in range(_CHUNK // 16):
                sl = pl.ds(o * 16, 16)
                m = msk_v[h, c, sl]
                idx_v[h, c, sl] = idx_v[h, c, sl] * m
                cnt = cnt + m
        cp0 = pltpu.async_copy(table.at[idx_v.at[h, 0]], rows0_v, sem)
        cp1 = pltpu.async_copy(table.at[idx_v.at[h, 1]], rows1_v, sem)
        cp0.wait()
        cp1.wait()

        acc = jnp.zeros((16,), jnp.int32)

        def red(i, a):
            for j in range(4):
                r = i * 4 + j
                a = a ^ rows0_v[r, :] ^ rows1_v[r, :]
            return a

        acc = lax.fori_loop(0, _CHUNK // 4, red, acc)
        odd = jnp.sum(cnt) & 1
        acc = jnp.where(jnp.broadcast_to(odd == 1, (16,)), acc ^ row0, acc)
        out_v[h, :] = acc
        return carry

    lax.fori_loop(0, _HPT, hint, 0)
    pltpu.sync_copy(out_v, out.at[pl.ds(base, _HPT)])


@jax.jit
def _sc_call(table, idx, msk):
    mesh = plsc.VectorSubcoreMesh(core_axis_name="c", subcore_axis_name="s",
                                  num_cores=_NC, num_subcores=_NS)
    return pl.kernel(
        _sc_body,
        out_type=jax.ShapeDtypeStruct((_NUM_HINTS, _ROW), jnp.int32),
        mesh=mesh,
        scratch_types=[
            pltpu.VMEM((_HPT, _NCHUNK, _CHUNK), jnp.int32),
            pltpu.VMEM((_HPT, _NCHUNK, _CHUNK), jnp.int32),
            pltpu.VMEM((_CHUNK, _ROW), jnp.int32),
            pltpu.VMEM((_CHUNK, _ROW), jnp.int32),
            pltpu.VMEM((_HPT, _ROW), jnp.int32),
            pltpu.VMEM((_ROW,), jnp.int32),
            pltpu.SemaphoreType.DMA,
        ],
    )(table, idx, msk)


def kernel(entries, padded_indices, valid_mask):
    e32 = lax.bitcast_convert_type(entries, jnp.int32)
    e32 = e32.reshape(_NUM_ENTRIES, 10)
    table = jnp.zeros((_NUM_ENTRIES, _ROW), jnp.int32).at[:, :10].set(e32)
    idx = padded_indices.astype(jnp.int32).reshape(_NUM_HINTS, _NCHUNK, _CHUNK)
    msk = valid_mask.astype(jnp.int32).reshape(_NUM_HINTS, _NCHUNK, _CHUNK)
    out32 = _sc_call(table, idx, msk)
    lo = out32[:, :10].reshape(_NUM_HINTS, 5, 2)
    return lax.bitcast_convert_type(lo, jnp.int64)


# speedup vs baseline: 2.8229x; 2.8229x over previous
import jax
import jax.numpy as jnp
from jax import lax
from jax.experimental import pallas as pl
from jax.experimental.pallas import tpu as pltpu
from jax.experimental.pallas import tpu_sc as plsc

jax.config.update("jax_enable_x64", True)

_NUM_ENTRIES = 65536
_NUM_HINTS = 4096
_MAX_SUBSET = 256
_ROW = 16
_NC = 2
_NS = 16
_NW = _NC * _NS
_HPT = _NUM_HINTS // _NW
_CHUNK = 128
_NCHUNK = _MAX_SUBSET // _CHUNK
_ZERO_ROW = _NUM_ENTRIES
_TABLE_ROWS = _NUM_ENTRIES + 8


def _sc_body(table, idx, msk, out, idx_v, msk_v, rows0_v, rows1_v, out_v, sem):
    wid = lax.axis_index("s") * _NC + lax.axis_index("c")
    base = wid * _HPT
    pltpu.sync_copy(idx.at[pl.ds(base, _HPT)], idx_v)
    pltpu.sync_copy(msk.at[pl.ds(base, _HPT)], msk_v)

    def hint(h, carry):
        for c in range(_NCHUNK):
            ci = jnp.int32(c)
            for o in range(_CHUNK // 16):
                sl = pl.ds(o * 16, 16)
                m = msk_v[h, ci, sl]
                ii = idx_v[h, ci, sl]
                idx_v[h, ci, sl] = ii * m + (1 - m) * jnp.int32(_ZERO_ROW)
        cp0 = pltpu.async_copy(table.at[idx_v.at[h, jnp.int32(0)]], rows0_v, sem)
        cp1 = pltpu.async_copy(table.at[idx_v.at[h, jnp.int32(1)]], rows1_v, sem)
        cp0.wait()
        cp1.wait()

        acc = jnp.zeros((16,), jnp.int32)

        def red(i, a):
            for j in range(4):
                r = i * 4 + j
                a = a ^ rows0_v[r, :] ^ rows1_v[r, :]
            return a

        acc = lax.fori_loop(jnp.int32(0), jnp.int32(_CHUNK // 4), red, acc)
        out_v[h, :] = acc
        return carry

    lax.fori_loop(jnp.int32(0), jnp.int32(_HPT), hint, jnp.int32(0))
    pltpu.sync_copy(out_v, out.at[pl.ds(base, _HPT)])


@jax.jit
def _sc_call(table, idx, msk):
    mesh = plsc.VectorSubcoreMesh(core_axis_name="c", subcore_axis_name="s",
                                  num_cores=_NC, num_subcores=_NS)
    return pl.kernel(
        _sc_body,
        out_type=jax.ShapeDtypeStruct((_NUM_HINTS, _ROW), jnp.int32),
        mesh=mesh,
        compiler_params=pltpu.CompilerParams(use_tc_tiling_on_sc=False),
        scratch_types=[
            pltpu.VMEM((_HPT, _NCHUNK, _CHUNK), jnp.int32),
            pltpu.VMEM((_HPT, _NCHUNK, _CHUNK), jnp.int32),
            pltpu.VMEM((_CHUNK, _ROW), jnp.int32),
            pltpu.VMEM((_CHUNK, _ROW), jnp.int32),
            pltpu.VMEM((_HPT, _ROW), jnp.int32),
            pltpu.SemaphoreType.DMA,
        ],
    )(table, idx, msk)


def kernel(entries, padded_indices, valid_mask):
    e32 = lax.bitcast_convert_type(entries, jnp.int32)
    e32 = e32.reshape(_NUM_ENTRIES, 10)
    table = jnp.zeros((_TABLE_ROWS, _ROW), jnp.int32).at[:_NUM_ENTRIES, :10].set(e32)
    idx = padded_indices.astype(jnp.int32).reshape(_NUM_HINTS, _NCHUNK, _CHUNK)
    msk = valid_mask.astype(jnp.int32).reshape(_NUM_HINTS, _NCHUNK, _CHUNK)
    out32 = _sc_call(table, idx, msk)
    lo = out32[:, :10].reshape(_NUM_HINTS, 5, 2)
    return lax.bitcast_convert_type(lo, jnp.int64)
